# Initial kernel scaffold; baseline (speedup 1.0000x reference)
import functools

import jax
import jax.numpy as jnp
from jax import lax
from jax.experimental import pallas as pl
from jax.experimental.pallas import tpu as pltpu
from jax.experimental.pallas import tpu_sc as plsc

_LANES = 16
_NBINS = 10001
_NB_PAD = 10240
_CHUNK = 8192


def _phase1(preds, targets):
    n = preds.shape[0]
    info = plsc.get_sparse_core_info()
    nc, ns = info.num_cores, info.num_subcores
    nw = nc * ns
    per_tile = n // nw
    nchunks = per_tile // _CHUNK
    slice_w = _NB_PAD // ns
    mesh = plsc.VectorSubcoreMesh(core_axis_name="c", subcore_axis_name="s")

    @functools.partial(
        pl.kernel,
        out_type=jax.ShapeDtypeStruct((nc, 2, _NB_PAD), jnp.float32),
        mesh=mesh,
        scratch_types=[
            pltpu.VMEM((_CHUNK,), jnp.float32),
            pltpu.VMEM((_CHUNK,), jnp.float32),
            pltpu.VMEM((_NB_PAD,), jnp.float32),
            pltpu.VMEM((_NB_PAD,), jnp.float32),
            pltpu.VMEM_SHARED((ns, _NB_PAD), jnp.float32),
            pltpu.VMEM_SHARED((ns, _NB_PAD), jnp.float32),
            pltpu.VMEM((ns, slice_w), jnp.float32),
            pltpu.VMEM((ns, slice_w), jnp.float32),
            pltpu.VMEM((slice_w,), jnp.float32),
            pltpu.VMEM((slice_w,), jnp.float32),
        ],
    )
    def k(preds_hbm, targets_hbm, out_hbm, pbuf, tbuf, hpos, hall,
          shpos, shall, gpos, gall, apos, aall):
        cid = lax.axis_index("c")
        sid = lax.axis_index("s")
        wid = sid * nc + cid

        zeros = jnp.zeros((_LANES,), jnp.float32)
        ones = jnp.ones((_LANES,), jnp.float32)

        def zbody(i, _):
            ds = pl.ds(i * _LANES, _LANES)
            hpos[ds] = zeros
            hall[ds] = zeros
            return 0

        lax.fori_loop(0, _NB_PAD // _LANES, zbody, 0)

        base = wid * per_tile

        def cbody(j, _):
            off = base + j * _CHUNK
            pltpu.sync_copy(preds_hbm.at[pl.ds(off, _CHUNK)], pbuf)
            pltpu.sync_copy(targets_hbm.at[pl.ds(off, _CHUNK)], tbuf)

            def vbody(i, _):
                ds = pl.ds(i * _LANES, _LANES)
                p = pbuf[ds]
                t = tbuf[ds]
                s = 1.0 / (1.0 + jnp.exp(-p))
                b = (10000.0 * s).astype(jnp.int32)
                pos = jnp.where(t >= 0.5, 1.0, 0.0)
                plsc.addupdate_scatter(hpos, [b], pos)
                plsc.addupdate_scatter(hall, [b], ones)
                return 0

            lax.fori_loop(0, _CHUNK // _LANES, vbody, 0)
            return 0

        lax.fori_loop(0, nchunks, cbody, 0)

        pltpu.sync_copy(hpos, shpos.at[sid])
        pltpu.sync_copy(hall, shall.at[sid])
        plsc.subcore_barrier()

        colbase = sid * slice_w
        for t in range(ns):
            pltpu.sync_copy(shpos.at[t, pl.ds(colbase, slice_w)], gpos.at[t])
            pltpu.sync_copy(shall.at[t, pl.ds(colbase, slice_w)], gall.at[t])

        def rbody(v, _):
            ds = pl.ds(v * _LANES, _LANES)
            sp = gpos[0, ds]
            sa = gall[0, ds]
            for t in range(1, ns):
                sp = sp + gpos[t, ds]
                sa = sa + gall[t, ds]
            apos[ds] = sp
            aall[ds] = sa
            return 0

        lax.fori_loop(0, slice_w // _LANES, rbody, 0)

        pltpu.sync_copy(apos, out_hbm.at[cid, 0, pl.ds(colbase, slice_w)])
        pltpu.sync_copy(aall, out_hbm.at[cid, 1, pl.ds(colbase, slice_w)])

    return k(preds, targets)


def _phase2(part, total):
    nc = part.shape[0]
    mesh = plsc.VectorSubcoreMesh(core_axis_name="c", subcore_axis_name="s")
    nv = _NB_PAD // _LANES

    @functools.partial(
        pl.kernel,
        out_type=jax.ShapeDtypeStruct((_LANES,), jnp.float32),
        mesh=mesh,
        scratch_types=[
            pltpu.VMEM((nc, 2, _NB_PAD), jnp.float32),
            pltpu.VMEM((_LANES,), jnp.float32),
        ],
    )
    def k(part_hbm, out_hbm, buf, obuf):
        cid = lax.axis_index("c")
        sid = lax.axis_index("s")

        @pl.when(jnp.logical_and(cid == 0, sid == 0))
        def _():
            pltpu.sync_copy(part_hbm, buf)

            def tbody(i, acc):
                ds = pl.ds(i * _LANES, _LANES)
                tp = buf[0, 0, ds]
                for c in range(1, nc):
                    tp = tp + buf[c, 0, ds]
                return acc + jnp.sum(tp)

            tp_tot = lax.fori_loop(0, nv, tbody, 0.0)
            inv_p = 1.0 / tp_tot
            inv_n = 1.0 / (total - tp_tot)

            def bodyB(i, carry):
                ctp, cal, m = carry
                ds = pl.ds(i * _LANES, _LANES)
                tp = buf[0, 0, ds]
                al = buf[0, 1, ds]
                for c in range(1, nc):
                    tp = tp + buf[c, 0, ds]
                    al = al + buf[c, 1, ds]
                tpc = plsc.cumsum(tp) + ctp
                alc = plsc.cumsum(al) + cal
                fpc = alc - tpc
                d = jnp.abs(tpc * inv_p - fpc * inv_n)
                m = jnp.maximum(m, jnp.max(d))
                return (ctp + jnp.sum(tp), cal + jnp.sum(al), m)

            _, _, m = lax.fori_loop(0, nv, bodyB, (0.0, 0.0, 0.0))
            obuf[...] = jnp.broadcast_to(m, (_LANES,))
            pltpu.sync_copy(obuf, out_hbm)

    return k(part)


def kernel(preds, targets):
    part = _phase1(preds, targets)
    ks = _phase2(part, float(preds.shape[0]))
    return ks[0]


# speedup vs baseline: 26.8699x; 26.8699x over previous
import functools

import jax
import jax.numpy as jnp
from jax import lax
from jax.experimental import pallas as pl
from jax.experimental.pallas import tpu as pltpu
from jax.experimental.pallas import tpu_sc as plsc

_LANES = 16
_NBINS = 10001
_NB_PAD = 10240
_CHUNK = 8192


def _phase1(preds, targets):
    n = preds.shape[0]
    info = plsc.get_sparse_core_info()
    nc, ns = info.num_cores, info.num_subcores
    nw = nc * ns
    per_tile = n // nw
    nchunks = per_tile // _CHUNK
    slice_w = _NB_PAD // ns
    mesh = plsc.VectorSubcoreMesh(core_axis_name="c", subcore_axis_name="s")

    @functools.partial(
        pl.kernel,
        out_type=jax.ShapeDtypeStruct((nc, 2, _NB_PAD), jnp.float32),
        mesh=mesh,
        compiler_params=pltpu.CompilerParams(needs_layout_passes=False),
        scratch_types=[
            pltpu.VMEM((_CHUNK,), jnp.float32),
            pltpu.VMEM((_CHUNK,), jnp.float32),
            pltpu.VMEM((_NB_PAD,), jnp.float32),
            pltpu.VMEM((_NB_PAD,), jnp.float32),
            pltpu.VMEM_SHARED((ns, _NB_PAD), jnp.float32),
            pltpu.VMEM_SHARED((ns, _NB_PAD), jnp.float32),
            pltpu.VMEM((ns, slice_w), jnp.float32),
            pltpu.VMEM((ns, slice_w), jnp.float32),
            pltpu.VMEM((slice_w,), jnp.float32),
            pltpu.VMEM((slice_w,), jnp.float32),
        ],
    )
    def k(preds_hbm, targets_hbm, out_hbm, pbuf, tbuf, hpos, hall,
          shpos, shall, gpos, gall, apos, aall):
        cid = lax.axis_index("c")
        sid = lax.axis_index("s")
        wid = sid * nc + cid

        zeros = jnp.zeros((_LANES,), jnp.float32)
        ones = jnp.ones((_LANES,), jnp.float32)

        def zbody(i, _):
            ds = pl.ds(i * _LANES, _LANES)
            hpos[ds] = zeros
            hall[ds] = zeros
            return 0

        lax.fori_loop(0, _NB_PAD // _LANES, zbody, 0)

        base = wid * per_tile

        def cbody(j, _):
            off = base + j * _CHUNK
            pltpu.sync_copy(preds_hbm.at[pl.ds(off, _CHUNK)], pbuf)
            pltpu.sync_copy(targets_hbm.at[pl.ds(off, _CHUNK)], tbuf)

            def vbody(i, _):
                ds = pl.ds(i * _LANES, _LANES)
                p = pbuf[ds]
                t = tbuf[ds]
                s = 1.0 / (1.0 + jnp.exp(-p))
                b = (10000.0 * s).astype(jnp.int32)
                pos = jnp.where(t >= 0.5, 1.0, 0.0)
                plsc.addupdate_scatter(hpos, [b], pos)
                plsc.addupdate_scatter(hall, [b], ones)
                return 0

            lax.fori_loop(0, _CHUNK // _LANES, vbody, 0)
            return 0

        lax.fori_loop(0, nchunks, cbody, 0)

        pltpu.sync_copy(hpos, shpos.at[sid])
        pltpu.sync_copy(hall, shall.at[sid])
        plsc.subcore_barrier()

        colbase = sid * slice_w
        for t in range(ns):
            pltpu.sync_copy(shpos.at[t, pl.ds(colbase, slice_w)], gpos.at[t])
            pltpu.sync_copy(shall.at[t, pl.ds(colbase, slice_w)], gall.at[t])

        def rbody(v, _):
            ds = pl.ds(v * _LANES, _LANES)
            sp = gpos[0, ds]
            sa = gall[0, ds]
            for t in range(1, ns):
                sp = sp + gpos[t, ds]
                sa = sa + gall[t, ds]
            apos[ds] = sp
            aall[ds] = sa
            return 0

        lax.fori_loop(0, slice_w // _LANES, rbody, 0)

        pltpu.sync_copy(apos, out_hbm.at[cid, 0, pl.ds(colbase, slice_w)])
        pltpu.sync_copy(aall, out_hbm.at[cid, 1, pl.ds(colbase, slice_w)])

    return k(preds, targets)


def _phase2(part, total):
    nc = part.shape[0]
    mesh = plsc.VectorSubcoreMesh(core_axis_name="c", subcore_axis_name="s")
    nv = _NB_PAD // _LANES

    @functools.partial(
        pl.kernel,
        out_type=jax.ShapeDtypeStruct((_LANES,), jnp.float32),
        mesh=mesh,
        compiler_params=pltpu.CompilerParams(needs_layout_passes=False),
        scratch_types=[
            pltpu.VMEM((nc, 2, _NB_PAD), jnp.float32),
            pltpu.VMEM((_LANES,), jnp.float32),
        ],
    )
    def k(part_hbm, out_hbm, buf, obuf):
        cid = lax.axis_index("c")
        sid = lax.axis_index("s")

        @pl.when(jnp.logical_and(cid == 0, sid == 0))
        def _():
            pltpu.sync_copy(part_hbm, buf)

            def tbody(i, acc):
                ds = pl.ds(i * _LANES, _LANES)
                tp = buf[0, 0, ds]
                for c in range(1, nc):
                    tp = tp + buf[c, 0, ds]
                return acc + jnp.sum(tp)

            tp_tot = lax.fori_loop(0, nv, tbody, 0.0)
            inv_p = 1.0 / jnp.broadcast_to(tp_tot, (_LANES,))
            inv_n = 1.0 / jnp.broadcast_to(total - tp_tot, (_LANES,))

            def bodyB(i, carry):
                ctp, cal, m = carry
                ds = pl.ds(i * _LANES, _LANES)
                tp = buf[0, 0, ds]
                al = buf[0, 1, ds]
                for c in range(1, nc):
                    tp = tp + buf[c, 0, ds]
                    al = al + buf[c, 1, ds]
                tpc = plsc.cumsum(tp) + ctp
                alc = plsc.cumsum(al) + cal
                fpc = alc - tpc
                d = jnp.abs(tpc * inv_p - fpc * inv_n)
                m = jnp.maximum(m, jnp.max(d))
                return (ctp + jnp.sum(tp), cal + jnp.sum(al), m)

            _, _, m = lax.fori_loop(0, nv, bodyB, (0.0, 0.0, 0.0))
            obuf[...] = jnp.broadcast_to(m, (_LANES,))
            pltpu.sync_copy(obuf, out_hbm)

    return k(part)


def kernel(preds, targets):
    part = _phase1(preds, targets)
    ks = _phase2(part, float(preds.shape[0]))
    return ks[0]


# speedup vs baseline: 30.6925x; 1.1423x over previous
import functools

import jax
import jax.numpy as jnp
from jax import lax
from jax.experimental import pallas as pl
from jax.experimental.pallas import tpu as pltpu
from jax.experimental.pallas import tpu_sc as plsc

_LANES = 16
_NBINS = 10001
_NB_PAD = 10240
_CHUNK = 8192
_UNROLL = 8


def _phase1(preds, targets):
    n = preds.shape[0]
    info = plsc.get_sparse_core_info()
    nc, ns = info.num_cores, info.num_subcores
    nw = nc * ns
    per_tile = n // nw
    nchunks = per_tile // _CHUNK
    slice_w = _NB_PAD // ns
    mesh = plsc.VectorSubcoreMesh(core_axis_name="c", subcore_axis_name="s")

    @functools.partial(
        pl.kernel,
        out_type=jax.ShapeDtypeStruct((nc, 2, _NB_PAD), jnp.float32),
        mesh=mesh,
        compiler_params=pltpu.CompilerParams(needs_layout_passes=False),
        scratch_types=[
            pltpu.VMEM((_CHUNK,), jnp.float32),
            pltpu.VMEM((_CHUNK,), jnp.float32),
            pltpu.VMEM((_CHUNK,), jnp.float32),
            pltpu.VMEM((_CHUNK,), jnp.float32),
            pltpu.VMEM((2 * _NB_PAD,), jnp.float32),
            pltpu.VMEM_SHARED((ns, 2 * _NB_PAD), jnp.float32),
            pltpu.VMEM((ns, slice_w), jnp.float32),
            pltpu.VMEM((ns, slice_w), jnp.float32),
            pltpu.VMEM((slice_w,), jnp.float32),
            pltpu.VMEM((slice_w,), jnp.float32),
            pltpu.SemaphoreType.DMA,
            pltpu.SemaphoreType.DMA,
            pltpu.SemaphoreType.DMA,
            pltpu.SemaphoreType.DMA,
        ],
    )
    def k(preds_hbm, targets_hbm, out_hbm, pbuf0, pbuf1, tbuf0, tbuf1,
          hist, shared, gneg, gpos, aneg, apos,
          psem0, psem1, tsem0, tsem1):
        cid = lax.axis_index("c")
        sid = lax.axis_index("s")
        wid = sid * nc + cid

        pbufs = (pbuf0, pbuf1)
        tbufs = (tbuf0, tbuf1)
        psems = (psem0, psem1)
        tsems = (tsem0, tsem1)

        zeros = jnp.zeros((_LANES,), jnp.float32)
        ones = jnp.ones((_LANES,), jnp.float32)

        def zbody(i, _):
            hist[pl.ds(i * _LANES, _LANES)] = zeros
            return 0

        lax.fori_loop(0, 2 * _NB_PAD // _LANES, zbody, 0, unroll=8)

        base = wid * per_tile

        for b in range(2):
            off = base + b * _CHUNK
            pltpu.async_copy(preds_hbm.at[pl.ds(off, _CHUNK)], pbufs[b], psems[b])
            pltpu.async_copy(targets_hbm.at[pl.ds(off, _CHUNK)], tbufs[b], tsems[b])

        def cbody(jj, _):
            for b in range(2):
                j = jj * 2 + b
                pb, tb = pbufs[b], tbufs[b]
                pltpu.make_async_copy(
                    preds_hbm.at[pl.ds(0, _CHUNK)], pb, psems[b]).wait()
                pltpu.make_async_copy(
                    targets_hbm.at[pl.ds(0, _CHUNK)], tb, tsems[b]).wait()

                def vbody(i, _, pb=pb, tb=tb):
                    ib = i * (_LANES * _UNROLL)
                    for u in range(_UNROLL):
                        ds = pl.ds(ib + u * _LANES, _LANES)
                        p = pb[ds]
                        t = tb[ds]
                        s = 1.0 / (1.0 + jnp.exp(-p))
                        bn = (10000.0 * s).astype(jnp.int32)
                        half = jnp.where(t >= 0.5, _NB_PAD, 0)
                        plsc.addupdate_scatter(hist, [bn + half], ones)
                    return 0

                lax.fori_loop(0, _CHUNK // (_LANES * _UNROLL), vbody, 0)

                nxt = j + 2

                @pl.when(nxt < nchunks)
                def _(b=b, nxt=nxt, pb=pb, tb=tb):
                    off = base + nxt * _CHUNK
                    pltpu.async_copy(
                        preds_hbm.at[pl.ds(off, _CHUNK)], pb, psems[b])
                    pltpu.async_copy(
                        targets_hbm.at[pl.ds(off, _CHUNK)], tb, tsems[b])
            return 0

        lax.fori_loop(0, nchunks // 2, cbody, 0)

        pltpu.sync_copy(hist, shared.at[sid])
        plsc.subcore_barrier()

        colbase = sid * slice_w
        for t in range(ns):
            pltpu.sync_copy(shared.at[t, pl.ds(colbase, slice_w)], gneg.at[t])
            pltpu.sync_copy(
                shared.at[t, pl.ds(_NB_PAD + colbase, slice_w)], gpos.at[t])

        def rbody(v, _):
            ds = pl.ds(v * _LANES, _LANES)
            sn = gneg[0, ds]
            sp = gpos[0, ds]
            for t in range(1, ns):
                sn = sn + gneg[t, ds]
                sp = sp + gpos[t, ds]
            aneg[ds] = sn
            apos[ds] = sp
            return 0

        lax.fori_loop(0, slice_w // _LANES, rbody, 0)

        pltpu.sync_copy(aneg, out_hbm.at[cid, 0, pl.ds(colbase, slice_w)])
        pltpu.sync_copy(apos, out_hbm.at[cid, 1, pl.ds(colbase, slice_w)])

    return k(preds, targets)


def _phase2(part, total):
    nc = part.shape[0]
    mesh = plsc.VectorSubcoreMesh(core_axis_name="c", subcore_axis_name="s")
    nv = _NB_PAD // _LANES

    @functools.partial(
        pl.kernel,
        out_type=jax.ShapeDtypeStruct((_LANES,), jnp.float32),
        mesh=mesh,
        compiler_params=pltpu.CompilerParams(needs_layout_passes=False),
        scratch_types=[
            pltpu.VMEM((nc, 2, _NB_PAD), jnp.float32),
            pltpu.VMEM((_LANES,), jnp.float32),
        ],
    )
    def k(part_hbm, out_hbm, buf, obuf):
        cid = lax.axis_index("c")
        sid = lax.axis_index("s")

        @pl.when(jnp.logical_and(cid == 0, sid == 0))
        def _():
            pltpu.sync_copy(part_hbm, buf)

            def tbody(i, acc):
                ds = pl.ds(i * _LANES, _LANES)
                tp = buf[0, 1, ds]
                for c in range(1, nc):
                    tp = tp + buf[c, 1, ds]
                return acc + jnp.sum(tp)

            tp_tot = lax.fori_loop(0, nv, tbody, 0.0)
            inv_p = 1.0 / jnp.broadcast_to(tp_tot, (_LANES,))
            inv_n = 1.0 / jnp.broadcast_to(total - tp_tot, (_LANES,))

            def bodyB(i, carry):
                ctp, cfp, m = carry
                ds = pl.ds(i * _LANES, _LANES)
                fp = buf[0, 0, ds]
                tp = buf[0, 1, ds]
                for c in range(1, nc):
                    fp = fp + buf[c, 0, ds]
                    tp = tp + buf[c, 1, ds]
                tpc = plsc.cumsum(tp) + ctp
                fpc = plsc.cumsum(fp) + cfp
                d = jnp.abs(tpc * inv_p - fpc * inv_n)
                m = jnp.maximum(m, jnp.max(d))
                return (ctp + jnp.sum(tp), cfp + jnp.sum(fp), m)

            _, _, m = lax.fori_loop(0, nv, bodyB, (0.0, 0.0, 0.0))
            obuf[...] = jnp.broadcast_to(m, (_LANES,))
            pltpu.sync_copy(obuf, out_hbm)

    return k(part)


def kernel(preds, targets):
    part = _phase1(preds, targets)
    ks = _phase2(part, float(preds.shape[0]))
    return ks[0]


# speedup vs baseline: 127.6589x; 4.1593x over previous
import functools

import jax
import jax.numpy as jnp
from jax import lax
from jax.experimental import pallas as pl
from jax.experimental.pallas import tpu as pltpu
from jax.experimental.pallas import tpu_sc as plsc

_LANES = 16
_NBINS = 10001
_NB_PAD = 10240
_CHUNK = 16384
_UNROLL = 16


def _phase1(preds, targets):
    n = preds.shape[0]
    info = plsc.get_sparse_core_info()
    nc, ns = info.num_cores, info.num_subcores
    nw = nc * ns
    per_tile = n // nw
    nchunks = per_tile // _CHUNK
    slice_w = _NB_PAD // ns
    mesh = plsc.VectorSubcoreMesh(core_axis_name="c", subcore_axis_name="s")

    @functools.partial(
        pl.kernel,
        out_type=[
            jax.ShapeDtypeStruct((nc, 2, _NB_PAD), jnp.float32),
            jax.ShapeDtypeStruct((nc * 2 * ns * _LANES,), jnp.float32),
        ],
        mesh=mesh,
        compiler_params=pltpu.CompilerParams(needs_layout_passes=False),
        scratch_types=[
            pltpu.VMEM((_CHUNK,), jnp.float32),
            pltpu.VMEM((_CHUNK,), jnp.float32),
            pltpu.VMEM((_CHUNK,), jnp.float32),
            pltpu.VMEM((_CHUNK,), jnp.float32),
            pltpu.VMEM((2 * _NB_PAD,), jnp.float32),
            pltpu.VMEM_SHARED((ns, 2 * _NB_PAD), jnp.float32),
            pltpu.VMEM((ns, slice_w), jnp.float32),
            pltpu.VMEM((ns, slice_w), jnp.float32),
            pltpu.VMEM((slice_w,), jnp.float32),
            pltpu.VMEM((slice_w,), jnp.float32),
            pltpu.VMEM((_LANES,), jnp.float32),
            pltpu.SemaphoreType.DMA,
            pltpu.SemaphoreType.DMA,
            pltpu.SemaphoreType.DMA,
            pltpu.SemaphoreType.DMA,
            pltpu.SemaphoreType.DMA,
        ],
    )
    def k(preds_hbm, targets_hbm, out_hbm, tots_hbm, pbuf0, pbuf1, tbuf0,
          tbuf1, hist, shared, gneg, gpos, aneg, apos, tmp,
          psem0, psem1, tsem0, tsem1, gsem):
        cid = lax.axis_index("c")
        sid = lax.axis_index("s")
        wid = sid * nc + cid

        pbufs = (pbuf0, pbuf1)
        tbufs = (tbuf0, tbuf1)
        psems = (psem0, psem1)
        tsems = (tsem0, tsem1)

        zeros = jnp.zeros((_LANES,), jnp.float32)
        ones = jnp.ones((_LANES,), jnp.float32)

        @plsc.parallel_loop(0, 2 * _NB_PAD // _LANES, unroll=8)
        def _(i):
            hist[pl.ds(i * _LANES, _LANES)] = zeros

        base = wid * per_tile

        for b in range(2):
            off = base + b * _CHUNK
            pltpu.async_copy(preds_hbm.at[pl.ds(off, _CHUNK)], pbufs[b], psems[b])
            pltpu.async_copy(targets_hbm.at[pl.ds(off, _CHUNK)], tbufs[b], tsems[b])

        def cbody(jj, _):
            for b in range(2):
                j = jj * 2 + b
                pb, tb = pbufs[b], tbufs[b]
                pltpu.make_async_copy(
                    preds_hbm.at[pl.ds(0, _CHUNK)], pb, psems[b]).wait()
                pltpu.make_async_copy(
                    targets_hbm.at[pl.ds(0, _CHUNK)], tb, tsems[b]).wait()

                @plsc.parallel_loop(0, _CHUNK // _LANES, unroll=_UNROLL)
                def _(i, pb=pb, tb=tb):
                    ds = pl.ds(i * _LANES, _LANES)
                    p = pb[ds]
                    t = tb[ds]
                    s = 1.0 / (1.0 + jnp.exp(-p))
                    bn = (10000.0 * s).astype(jnp.int32)
                    half = jnp.where(t >= 0.5, _NB_PAD, 0)
                    plsc.addupdate_scatter(hist, [bn + half], ones)

                nxt = j + 2

                @pl.when(nxt < nchunks)
                def _(b=b, pb=pb, tb=tb, nxt=nxt):
                    off = base + nxt * _CHUNK
                    pltpu.async_copy(
                        preds_hbm.at[pl.ds(off, _CHUNK)], pb, psems[b])
                    pltpu.async_copy(
                        targets_hbm.at[pl.ds(off, _CHUNK)], tb, tsems[b])
            return 0

        lax.fori_loop(0, nchunks // 2, cbody, 0)

        pltpu.sync_copy(hist, shared.at[sid])
        plsc.subcore_barrier()

        colbase = sid * slice_w
        for t in range(ns):
            pltpu.async_copy(
                shared.at[t, pl.ds(colbase, slice_w)], gneg.at[t], gsem)
            pltpu.async_copy(
                shared.at[t, pl.ds(_NB_PAD + colbase, slice_w)], gpos.at[t], gsem)
        for t in range(ns):
            pltpu.make_async_copy(
                shared.at[t, pl.ds(colbase, slice_w)], gneg.at[t], gsem).wait()
            pltpu.make_async_copy(
                shared.at[t, pl.ds(colbase, slice_w)], gpos.at[t], gsem).wait()

        @plsc.parallel_loop(0, slice_w // _LANES, unroll=4)
        def _(v):
            ds = pl.ds(v * _LANES, _LANES)
            sn = gneg[0, ds]
            sp = gpos[0, ds]
            for t in range(1, ns):
                sn = sn + gneg[t, ds]
                sp = sp + gpos[t, ds]
            aneg[ds] = sn
            apos[ds] = sp

        pltpu.sync_copy(aneg, out_hbm.at[cid, 0, pl.ds(colbase, slice_w)])
        pltpu.sync_copy(apos, out_hbm.at[cid, 1, pl.ds(colbase, slice_w)])

        def sbody(v, carry):
            sn, sp = carry
            ds = pl.ds(v * _LANES, _LANES)
            return (sn + aneg[ds], sp + apos[ds])

        sn, sp = lax.fori_loop(0, slice_w // _LANES, sbody, (zeros, zeros))
        tmp[...] = jnp.broadcast_to(jnp.sum(sn), (_LANES,))
        pltpu.sync_copy(
            tmp, tots_hbm.at[pl.ds(((cid * 2 + 0) * ns + sid) * _LANES, _LANES)])
        tmp[...] = jnp.broadcast_to(jnp.sum(sp), (_LANES,))
        pltpu.sync_copy(
            tmp, tots_hbm.at[pl.ds(((cid * 2 + 1) * ns + sid) * _LANES, _LANES)])

    return k(preds, targets)


def _phase2(part, tots):
    nc = part.shape[0]
    info = plsc.get_sparse_core_info()
    ns = info.num_subcores
    slice_w = _NB_PAD // ns
    nv = slice_w // _LANES
    mesh = plsc.VectorSubcoreMesh(core_axis_name="c", subcore_axis_name="s")

    @functools.partial(
        pl.kernel,
        out_type=[
            jax.ShapeDtypeStruct((_LANES,), jnp.float32),
            jax.ShapeDtypeStruct((ns * _LANES,), jnp.float32),
        ],
        mesh=mesh,
        compiler_params=pltpu.CompilerParams(needs_layout_passes=False),
        scratch_types=[
            pltpu.VMEM((nc, 2, slice_w), jnp.float32),
            pltpu.VMEM((slice_w,), jnp.float32),
            pltpu.VMEM((slice_w,), jnp.float32),
            pltpu.VMEM((nc * 2 * ns * _LANES,), jnp.float32),
            pltpu.VMEM((ns * _LANES,), jnp.float32),
            pltpu.VMEM((_LANES,), jnp.float32),
            pltpu.VMEM((_LANES,), jnp.float32),
        ],
    )
    def k(part_hbm, tots_hbm, out_hbm, mx_hbm, vbuf, fsl, tsl, ttot,
          gmx, tmp, obuf):
        cid = lax.axis_index("c")
        sid = lax.axis_index("s")

        @pl.when(cid == 0)
        def _():
            colbase = sid * slice_w
            for c in range(nc):
                for h in range(2):
                    pltpu.sync_copy(
                        part_hbm.at[c, h, pl.ds(colbase, slice_w)],
                        vbuf.at[c, h])
            pltpu.sync_copy(tots_hbm, ttot)

            zeros = jnp.zeros((_LANES,), jnp.float32)

            @plsc.parallel_loop(0, nv, unroll=4)
            def _(i):
                ds = pl.ds(i * _LANES, _LANES)
                fp = vbuf[0, 0, ds]
                tp = vbuf[0, 1, ds]
                for c in range(1, nc):
                    fp = fp + vbuf[c, 0, ds]
                    tp = tp + vbuf[c, 1, ds]
                fsl[ds] = fp
                tsl[ds] = tp

            sidv = jnp.broadcast_to(sid, (_LANES,))
            pref_t = zeros
            pref_f = zeros
            tot_t = zeros
            tot_f = zeros
            for c in range(nc):
                for j in range(ns):
                    rowf = ttot[pl.ds(((c * 2 + 0) * ns + j) * _LANES, _LANES)]
                    rowt = ttot[pl.ds(((c * 2 + 1) * ns + j) * _LANES, _LANES)]
                    before = jnp.broadcast_to(jnp.int32(j), (_LANES,)) < sidv
                    pref_f = pref_f + jnp.where(before, rowf, zeros)
                    pref_t = pref_t + jnp.where(before, rowt, zeros)
                    tot_f = tot_f + rowf
                    tot_t = tot_t + rowt

            inv_p = 1.0 / tot_t
            inv_n = 1.0 / tot_f

            def kbody(i, carry):
                ct, cf, m = carry
                ds = pl.ds(i * _LANES, _LANES)
                tp = tsl[ds]
                fp = fsl[ds]
                tpc = plsc.cumsum(tp) + ct
                fpc = plsc.cumsum(fp) + cf
                d = jnp.abs(tpc * inv_p - fpc * inv_n)
                m = jnp.maximum(m, jnp.max(d))
                return (ct + jnp.sum(tp), cf + jnp.sum(fp), m)

            _, _, m = lax.fori_loop(0, nv, kbody, (pref_t, pref_f, 0.0))

            tmp[...] = jnp.broadcast_to(m, (_LANES,))
            pltpu.sync_copy(tmp, mx_hbm.at[pl.ds(sid * _LANES, _LANES)])
            plsc.subcore_barrier()

            @pl.when(sid == 0)
            def _():
                pltpu.sync_copy(mx_hbm, gmx)
                mm = gmx[pl.ds(0, _LANES)]
                for j in range(1, ns):
                    mm = jnp.maximum(mm, gmx[pl.ds(j * _LANES, _LANES)])
                obuf[...] = mm
                pltpu.sync_copy(obuf, out_hbm)

    return k(part, tots)


def kernel(preds, targets):
    part, tots = _phase1(preds, targets)
    ks, _ = _phase2(part, tots)
    return ks[0]
